# baseline (speedup 1.0000x reference)
import jax
import jax.numpy as jnp
import numpy as np
from jax.experimental import pallas as pl
from jax.experimental.pallas import tpu as pltpu

DIM = 128
HEADS = 4
HEAD_DIM = DIM // HEADS
NUM_SEG = 64
EPS = 1e-5
MEAN_RES_SCALE = 0.2
BLK = 2000
NEG = -1e30


def _fused(h_ref, brow_ref, bcol_ref, ci_ref, wqT_ref, lnqg_ref, lnqb_ref,
           wresT_ref, wkvT_ref, woutT_ref, lngc_ref, lnb_ref, scal_ref,
           out_ref, sum_s, cnt_s, max_s, q_s, mres_s, m_s, da_s,
           cj_s, wkvg_s, selaug_s, selt_s):
    p = pl.program_id(0)
    i = pl.program_id(1)
    nblk = pl.num_programs(1)
    h = h_ref[...]
    bcol = bcol_ref[...]
    brow = brow_ref[0]
    blk = bcol.shape[0]
    seg_c = jax.lax.broadcasted_iota(jnp.int32, (NUM_SEG, blk), 0)
    onehot_t = (seg_c == brow).astype(jnp.float32)
    s0 = bcol[0, 0]
    s1 = bcol[blk - 1, 0]

    @pl.when(jnp.logical_and(p == 0, i == 0))
    def _const_init():
        sum_s[...] = jnp.zeros_like(sum_s)
        cnt_s[...] = jnp.zeros_like(cnt_s)
        max_s[...] = jnp.full_like(max_s, -jnp.inf)
        r = jax.lax.broadcasted_iota(jnp.int32, (DIM, 2 * DIM), 0)
        c = jax.lax.broadcasted_iota(jnp.int32, (DIM, 2 * DIM), 1)
        cj_s[...] = jnp.where(c < DIM, (r == c).astype(jnp.float32), 0.0) \
            - (1.0 / DIM)
        sd = jax.lax.broadcasted_iota(jnp.int32, (DIM + 8, 8), 0)
        sh = jax.lax.broadcasted_iota(jnp.int32, (DIM + 8, 8), 1)
        lscale = scal_ref[0, 2] * (1.0 / np.sqrt(HEAD_DIM))
        sel = jnp.where(jnp.logical_and(sd < DIM, sd // HEAD_DIM == sh),
                        lscale, 0.0)
        sel = jnp.where(sd == DIM, scal_ref[0, 0], sel)
        selaug_s[...] = jnp.where(sd == DIM + 1, scal_ref[0, 1], sel)
        td = jax.lax.broadcasted_iota(jnp.int32, (8, DIM), 1) // HEAD_DIM
        th = jax.lax.broadcasted_iota(jnp.int32, (8, DIM), 0)
        selt_s[...] = (td == th).astype(jnp.float32)

    @pl.when(p == 0)
    def _phase_a():
        sum_s[...] += jnp.dot(onehot_t, h, preferred_element_type=jnp.float32)
        cnt_s[...] += jnp.sum(onehot_t, axis=1, keepdims=True)

        seg_rows = jax.lax.broadcasted_iota(jnp.int32, (NUM_SEG, 1), 0)

        def body(s, _):
            m = jnp.max(jnp.where(bcol == s, h, -jnp.inf), axis=0,
                        keepdims=True)
            max_s[...] = jnp.where(seg_rows == s,
                                   jnp.maximum(max_s[...], m), max_s[...])
            return 0

        jax.lax.fori_loop(s0, s1 + 1, body, 0)

    @pl.when(p == 1)
    def _phase_b():
        @pl.when(i == 0)
        def _mid():
            mean = sum_s[...] / cnt_s[...]
            qa = (jnp.dot(mean, wqT_ref[:DIM, :],
                          preferred_element_type=jnp.float32)
                  + jnp.dot(max_s[...], wqT_ref[DIM:, :],
                            preferred_element_type=jnp.float32))
            mu = jnp.mean(qa, axis=-1, keepdims=True)
            var = jnp.mean((qa - mu) ** 2, axis=-1, keepdims=True)
            q_s[...] = ((qa - mu) / jnp.sqrt(var + EPS) * lnqg_ref[...]
                        + lnqb_ref[...])
            mres_s[...] = MEAN_RES_SCALE * jnp.dot(
                mean, wresT_ref[...], preferred_element_type=jnp.float32)
            m_s[...] = jnp.full_like(m_s, NEG)
            da_s[...] = jnp.zeros_like(da_s)
            rows = jax.lax.broadcasted_iota(jnp.int32, (DIM + 8, 1), 0)
            wg = lngc_ref[...] * wkvT_ref[...]
            bkv = jnp.dot(lnb_ref[...], wkvT_ref[...],
                          preferred_element_type=jnp.float32)
            wkvg_s[...] = jnp.where(
                rows < DIM,
                jnp.pad(wg, ((0, 8), (0, 0))),
                jnp.where(rows == DIM,
                          jnp.broadcast_to(bkv, (DIM + 8, 2 * DIM)), 0.0))

        h_cm = jnp.dot(h, cj_s[...], preferred_element_type=jnp.float32)
        h_c = h_cm[:, :DIM]
        mu = h_cm[:, DIM:]
        msq = jnp.dot(h * h, jnp.full((DIM, DIM), 1.0 / DIM, jnp.float32),
                      preferred_element_type=jnp.float32)
        t = h_c * jax.lax.rsqrt(msq - mu * mu + EPS)
        ones8 = jnp.ones((blk, 8), jnp.float32)
        kv = jnp.dot(jnp.concatenate([t, ones8], axis=1), wkvg_s[...],
                     preferred_element_type=jnp.float32)
        k = kv[:, :DIM]
        v = kv[:, DIM:]

        seg_r = jax.lax.broadcasted_iota(jnp.int32, (blk, NUM_SEG), 1)
        onehot = (bcol == seg_r).astype(jnp.float32)
        qrows = jnp.dot(onehot, q_s[...], preferred_element_type=jnp.float32)

        lhs = jnp.concatenate([k * qrows, ci_ref[...]], axis=1)
        logit = jnp.dot(lhs, selaug_s[...],
                        preferred_element_type=jnp.float32)

        bmax = jnp.max(logit, axis=0, keepdims=True)
        seg8 = jax.lax.broadcasted_iota(jnp.int32, (NUM_SEG, 8), 0)
        present = jnp.logical_and(seg8 >= s0, seg8 <= s1)
        mo = m_s[...]
        mn = jnp.where(present, jnp.maximum(mo, bmax), mo)
        scale = jnp.exp(mo - mn)
        m_s[...] = mn

        mrows = jnp.dot(onehot, mn, preferred_element_type=jnp.float32)
        w = jnp.exp(logit - mrows)
        wb = jnp.dot(w, selt_s[...], preferred_element_type=jnp.float32)
        rhs = jnp.concatenate([wb * v, w], axis=1)
        scale_ext = jnp.concatenate(
            [jnp.dot(scale, selt_s[...], preferred_element_type=jnp.float32),
             scale], axis=1)
        da_s[...] = da_s[...] * scale_ext + jnp.dot(
            onehot_t, rhs, preferred_element_type=jnp.float32)

        @pl.when(i == nblk - 1)
        def _fin():
            denom = jnp.dot(da_s[:, DIM:], selt_s[...],
                            preferred_element_type=jnp.float32)
            g_attn = da_s[:, :DIM] / denom
            out_ref[...] = jnp.dot(g_attn, woutT_ref[...],
                                   preferred_element_type=jnp.float32) \
                + mres_s[...]


def kernel(h, batch, cdr_mask, iface_mask, Wk, Wv, Wq, Wres, Wout,
           ln_kv_g, ln_kv_b, ln_q_g, ln_q_b, cdr_bias, iface_bias, logit_scale):
    n = h.shape[0]
    grid = n // BLK
    assert grid * BLK == n

    batch = batch.astype(jnp.int32)
    brow = batch.reshape(grid, 1, BLK)
    bcol = batch.reshape(n, 1)
    ci = jnp.pad(jnp.stack([cdr_mask, iface_mask], axis=1).astype(jnp.float32),
                 ((0, 0), (0, 6)))
    wkvT = jnp.concatenate([Wk.T, Wv.T], axis=1)
    scal = jnp.stack([cdr_bias, iface_bias, logit_scale]).reshape(1, 3)

    cmap = lambda p, i: (0, 0)
    out = pl.pallas_call(
        _fused,
        grid=(2, grid),
        in_specs=[pl.BlockSpec((BLK, DIM), lambda p, i: (i, 0)),
                  pl.BlockSpec((1, 1, BLK), lambda p, i: (i, 0, 0)),
                  pl.BlockSpec((BLK, 1), lambda p, i: (i, 0)),
                  pl.BlockSpec((BLK, 8), lambda p, i: (i, 0)),
                  pl.BlockSpec((2 * DIM, DIM), cmap),
                  pl.BlockSpec((1, DIM), cmap),
                  pl.BlockSpec((1, DIM), cmap),
                  pl.BlockSpec((DIM, DIM), cmap),
                  pl.BlockSpec((DIM, 2 * DIM), cmap),
                  pl.BlockSpec((DIM, DIM), cmap),
                  pl.BlockSpec((DIM, 1), cmap),
                  pl.BlockSpec((1, DIM), cmap),
                  pl.BlockSpec((1, 3), cmap)],
        out_specs=pl.BlockSpec((NUM_SEG, DIM), cmap),
        out_shape=jax.ShapeDtypeStruct((NUM_SEG, DIM), jnp.float32),
        scratch_shapes=[pltpu.VMEM((NUM_SEG, DIM), jnp.float32),
                        pltpu.VMEM((NUM_SEG, DIM), jnp.float32),
                        pltpu.VMEM((NUM_SEG, DIM), jnp.float32),
                        pltpu.VMEM((NUM_SEG, DIM), jnp.float32),
                        pltpu.VMEM((NUM_SEG, DIM), jnp.float32),
                        pltpu.VMEM((NUM_SEG, 8), jnp.float32),
                        pltpu.VMEM((NUM_SEG, DIM + 8), jnp.float32),
                        pltpu.VMEM((DIM, 2 * DIM), jnp.float32),
                        pltpu.VMEM((DIM + 8, 2 * DIM), jnp.float32),
                        pltpu.VMEM((DIM + 8, 8), jnp.float32),
                        pltpu.VMEM((8, DIM), jnp.float32)],
        compiler_params=pltpu.CompilerParams(
            dimension_semantics=("arbitrary", "arbitrary")),
    )(h, brow, bcol, ci, Wq.T, ln_q_g.reshape(1, DIM), ln_q_b.reshape(1, DIM),
      Wres.T, wkvT, Wout.T, ln_kv_g.reshape(DIM, 1), ln_kv_b.reshape(1, DIM),
      scal)

    return out


# speedup vs baseline: 1.0459x; 1.0459x over previous
import jax
import jax.numpy as jnp
import numpy as np
from jax.experimental import pallas as pl
from jax.experimental.pallas import tpu as pltpu

DIM = 128
HEADS = 4
HEAD_DIM = DIM // HEADS
NUM_SEG = 64
EPS = 1e-5
MEAN_RES_SCALE = 0.2
BLK = 4000
NEG = -1e30


def _fused(h_ref, brow_ref, bcol_ref, ci_ref, wqT_ref, lnqg_ref, lnqb_ref,
           wresT_ref, wkvT_ref, woutT_ref, lngc_ref, lnb_ref, scal_ref,
           out_ref, sum_s, cnt_s, max_s, q_s, mres_s, m_s, da_s,
           cj_s, wkvg_s, selaug_s, selt_s):
    p = pl.program_id(0)
    i = pl.program_id(1)
    nblk = pl.num_programs(1)
    h = h_ref[...]
    bcol = bcol_ref[...]
    brow = brow_ref[0]
    blk = bcol.shape[0]
    seg_c = jax.lax.broadcasted_iota(jnp.int32, (NUM_SEG, blk), 0)
    onehot_t = (seg_c == brow).astype(jnp.float32)
    s0 = bcol[0, 0]
    s1 = bcol[blk - 1, 0]

    @pl.when(jnp.logical_and(p == 0, i == 0))
    def _const_init():
        sum_s[...] = jnp.zeros_like(sum_s)
        cnt_s[...] = jnp.zeros_like(cnt_s)
        max_s[...] = jnp.full_like(max_s, -jnp.inf)
        r = jax.lax.broadcasted_iota(jnp.int32, (DIM, 2 * DIM), 0)
        c = jax.lax.broadcasted_iota(jnp.int32, (DIM, 2 * DIM), 1)
        cj_s[...] = jnp.where(c < DIM, (r == c).astype(jnp.float32), 0.0) \
            - (1.0 / DIM)
        sd = jax.lax.broadcasted_iota(jnp.int32, (DIM + 8, 8), 0)
        sh = jax.lax.broadcasted_iota(jnp.int32, (DIM + 8, 8), 1)
        lscale = scal_ref[0, 2] * (1.0 / np.sqrt(HEAD_DIM))
        sel = jnp.where(jnp.logical_and(sd < DIM, sd // HEAD_DIM == sh),
                        lscale, 0.0)
        sel = jnp.where(sd == DIM, scal_ref[0, 0], sel)
        selaug_s[...] = jnp.where(sd == DIM + 1, scal_ref[0, 1], sel)
        td = jax.lax.broadcasted_iota(jnp.int32, (8, DIM), 1) // HEAD_DIM
        th = jax.lax.broadcasted_iota(jnp.int32, (8, DIM), 0)
        selt_s[...] = (td == th).astype(jnp.float32)

    @pl.when(p == 0)
    def _phase_a():
        sum_s[...] += jnp.dot(onehot_t, h, preferred_element_type=jnp.float32)
        cnt_s[...] += jnp.sum(onehot_t, axis=1, keepdims=True)

        seg_rows = jax.lax.broadcasted_iota(jnp.int32, (NUM_SEG, 1), 0)

        def body(s, _):
            m = jnp.max(jnp.where(bcol == s, h, -jnp.inf), axis=0,
                        keepdims=True)
            max_s[...] = jnp.where(seg_rows == s,
                                   jnp.maximum(max_s[...], m), max_s[...])
            return 0

        jax.lax.fori_loop(s0, s1 + 1, body, 0)

    @pl.when(p == 1)
    def _phase_b():
        @pl.when(i == 0)
        def _mid():
            mean = sum_s[...] / cnt_s[...]
            qa = (jnp.dot(mean, wqT_ref[:DIM, :],
                          preferred_element_type=jnp.float32)
                  + jnp.dot(max_s[...], wqT_ref[DIM:, :],
                            preferred_element_type=jnp.float32))
            mu = jnp.mean(qa, axis=-1, keepdims=True)
            var = jnp.mean((qa - mu) ** 2, axis=-1, keepdims=True)
            q_s[...] = ((qa - mu) / jnp.sqrt(var + EPS) * lnqg_ref[...]
                        + lnqb_ref[...])
            mres_s[...] = MEAN_RES_SCALE * jnp.dot(
                mean, wresT_ref[...], preferred_element_type=jnp.float32)
            m_s[...] = jnp.full_like(m_s, NEG)
            da_s[...] = jnp.zeros_like(da_s)
            rows = jax.lax.broadcasted_iota(jnp.int32, (DIM + 8, 1), 0)
            wg = lngc_ref[...] * wkvT_ref[...]
            bkv = jnp.dot(lnb_ref[...], wkvT_ref[...],
                          preferred_element_type=jnp.float32)
            wkvg_s[...] = jnp.where(
                rows < DIM,
                jnp.pad(wg, ((0, 8), (0, 0))),
                jnp.where(rows == DIM,
                          jnp.broadcast_to(bkv, (DIM + 8, 2 * DIM)), 0.0))

        h_cm = jnp.dot(h, cj_s[...], preferred_element_type=jnp.float32)
        h_c = h_cm[:, :DIM]
        mu = h_cm[:, DIM:]
        msq = jnp.dot(h * h, jnp.full((DIM, DIM), 1.0 / DIM, jnp.float32),
                      preferred_element_type=jnp.float32)
        t = h_c * jax.lax.rsqrt(msq - mu * mu + EPS)
        ones8 = jnp.ones((blk, 8), jnp.float32)
        kv = jnp.dot(jnp.concatenate([t, ones8], axis=1), wkvg_s[...],
                     preferred_element_type=jnp.float32)
        k = kv[:, :DIM]
        v = kv[:, DIM:]

        seg_r = jax.lax.broadcasted_iota(jnp.int32, (blk, NUM_SEG), 1)
        onehot = (bcol == seg_r).astype(jnp.float32)
        qrows = jnp.dot(onehot, q_s[...], preferred_element_type=jnp.float32)

        lhs = jnp.concatenate([k * qrows, ci_ref[...]], axis=1)
        logit = jnp.dot(lhs, selaug_s[...],
                        preferred_element_type=jnp.float32)

        bmax = jnp.max(logit, axis=0, keepdims=True)
        seg8 = jax.lax.broadcasted_iota(jnp.int32, (NUM_SEG, 8), 0)
        present = jnp.logical_and(seg8 >= s0, seg8 <= s1)
        mo = m_s[...]
        mn = jnp.where(present, jnp.maximum(mo, bmax), mo)
        scale = jnp.exp(mo - mn)
        m_s[...] = mn

        mrows = jnp.dot(onehot, mn, preferred_element_type=jnp.float32)
        w = jnp.exp(logit - mrows)
        wb = jnp.dot(w, selt_s[...], preferred_element_type=jnp.float32)
        rhs = jnp.concatenate([wb * v, w], axis=1)
        scale_ext = jnp.concatenate(
            [jnp.dot(scale, selt_s[...], preferred_element_type=jnp.float32),
             scale], axis=1)
        da_s[...] = da_s[...] * scale_ext + jnp.dot(
            onehot_t, rhs, preferred_element_type=jnp.float32)

        @pl.when(i == nblk - 1)
        def _fin():
            denom = jnp.dot(da_s[:, DIM:], selt_s[...],
                            preferred_element_type=jnp.float32)
            g_attn = da_s[:, :DIM] / denom
            out_ref[...] = jnp.dot(g_attn, woutT_ref[...],
                                   preferred_element_type=jnp.float32) \
                + mres_s[...]


def kernel(h, batch, cdr_mask, iface_mask, Wk, Wv, Wq, Wres, Wout,
           ln_kv_g, ln_kv_b, ln_q_g, ln_q_b, cdr_bias, iface_bias, logit_scale):
    n = h.shape[0]
    grid = n // BLK
    assert grid * BLK == n

    batch = batch.astype(jnp.int32)
    brow = batch.reshape(grid, 1, BLK)
    bcol = batch.reshape(n, 1)
    ci = jnp.pad(jnp.stack([cdr_mask, iface_mask], axis=1).astype(jnp.float32),
                 ((0, 0), (0, 6)))
    wkvT = jnp.concatenate([Wk.T, Wv.T], axis=1)
    scal = jnp.stack([cdr_bias, iface_bias, logit_scale]).reshape(1, 3)

    cmap = lambda p, i: (0, 0)
    out = pl.pallas_call(
        _fused,
        grid=(2, grid),
        in_specs=[pl.BlockSpec((BLK, DIM), lambda p, i: (i, 0)),
                  pl.BlockSpec((1, 1, BLK), lambda p, i: (i, 0, 0)),
                  pl.BlockSpec((BLK, 1), lambda p, i: (i, 0)),
                  pl.BlockSpec((BLK, 8), lambda p, i: (i, 0)),
                  pl.BlockSpec((2 * DIM, DIM), cmap),
                  pl.BlockSpec((1, DIM), cmap),
                  pl.BlockSpec((1, DIM), cmap),
                  pl.BlockSpec((DIM, DIM), cmap),
                  pl.BlockSpec((DIM, 2 * DIM), cmap),
                  pl.BlockSpec((DIM, DIM), cmap),
                  pl.BlockSpec((DIM, 1), cmap),
                  pl.BlockSpec((1, DIM), cmap),
                  pl.BlockSpec((1, 3), cmap)],
        out_specs=pl.BlockSpec((NUM_SEG, DIM), cmap),
        out_shape=jax.ShapeDtypeStruct((NUM_SEG, DIM), jnp.float32),
        scratch_shapes=[pltpu.VMEM((NUM_SEG, DIM), jnp.float32),
                        pltpu.VMEM((NUM_SEG, DIM), jnp.float32),
                        pltpu.VMEM((NUM_SEG, DIM), jnp.float32),
                        pltpu.VMEM((NUM_SEG, DIM), jnp.float32),
                        pltpu.VMEM((NUM_SEG, DIM), jnp.float32),
                        pltpu.VMEM((NUM_SEG, 8), jnp.float32),
                        pltpu.VMEM((NUM_SEG, DIM + 8), jnp.float32),
                        pltpu.VMEM((DIM, 2 * DIM), jnp.float32),
                        pltpu.VMEM((DIM + 8, 2 * DIM), jnp.float32),
                        pltpu.VMEM((DIM + 8, 8), jnp.float32),
                        pltpu.VMEM((8, DIM), jnp.float32)],
        compiler_params=pltpu.CompilerParams(
            dimension_semantics=("arbitrary", "arbitrary")),
    )(h, brow, bcol, ci, Wq.T, ln_q_g.reshape(1, DIM), ln_q_b.reshape(1, DIM),
      Wres.T, wkvT, Wout.T, ln_kv_g.reshape(DIM, 1), ln_kv_b.reshape(1, DIM),
      scal)

    return out
